# Initial kernel scaffold; baseline (speedup 1.0000x reference)
import jax
import jax.numpy as jnp
from jax.experimental import pallas as pl


def kernel(x, tok_table, pos_table):
    raise NotImplementedError("write your pallas kernel here")


# speedup vs baseline: 1.3304x; 1.3304x over previous
import jax
import jax.numpy as jnp
from jax.experimental import pallas as pl
from jax.experimental.pallas import tpu as pltpu
from jax.experimental.pallas import tpu_sc as plsc

VOCAB = 30522
SEQ = 128
DIM = 256
LANES = 16


def kernel(x, tok_table, pos_table):
    batch, seq = x.shape
    n = batch * seq
    idx = x.reshape(1, n).astype(jnp.int32)

    mesh = plsc.VectorSubcoreMesh(core_axis_name="core", subcore_axis_name="subcore")

    @pl.kernel(
        out_type=jax.ShapeDtypeStruct((n, DIM), jnp.float32),
        mesh=mesh,
        scratch_types=[
            pltpu.VMEM((SEQ, DIM), jnp.float32),
        ],
    )
    def emb_kernel(tok_hbm, i_hbm, pos_hbm, o_hbm, pos_v):
        pltpu.sync_copy(pos_hbm, pos_v)

        def body(i_vmem, o_vmem):
            pltpu.sync_copy(tok_hbm.at[i_vmem.at[0]], o_vmem)

            @pl.loop(0, SEQ)
            def _(r):
                for j in range(DIM // LANES):
                    slc = pl.ds(j * LANES, LANES)
                    o_vmem.at[r, slc][...] = (
                        o_vmem.at[r, slc][...] + pos_v.at[r, slc][...]
                    )

        pltpu.emit_pipeline(
            body,
            grid=(n // SEQ,),
            in_specs=[pl.BlockSpec((1, SEQ), index_map=lambda i: (0, i))],
            out_specs=[pl.BlockSpec((SEQ, DIM), index_map=lambda i: (i, 0))],
            core_axis_name=("core", "subcore"),
            dimension_semantics=(pltpu.PARALLEL,),
        )(i_hbm, o_hbm)

    out = emb_kernel(tok_table, idx, pos_table)
    return out.reshape(batch, seq, DIM)


# speedup vs baseline: 2.8088x; 2.1112x over previous
import jax
import jax.numpy as jnp
from jax import lax
from jax.experimental import pallas as pl
from jax.experimental.pallas import tpu as pltpu
from jax.experimental.pallas import tpu_sc as plsc

VOCAB = 30522
SEQ = 128
DIM = 256
LANES = 16
NC = 2
NS = 16
NW = NC * NS
WIN = 64
NBUF = 4


def kernel(x, tok_table, pos_table):
    batch, seq = x.shape
    n = batch * seq
    per_w = n // NW
    nwin = per_w // WIN
    idx_flat = x.reshape(n).astype(jnp.int32)

    mesh = plsc.VectorSubcoreMesh(core_axis_name="core", subcore_axis_name="subcore")

    @pl.kernel(
        out_type=jax.ShapeDtypeStruct((n, DIM), jnp.float32),
        mesh=mesh,
        scratch_types=(
            [
                pltpu.VMEM((SEQ, DIM), jnp.float32),
                pltpu.VMEM((per_w,), jnp.int32),
            ]
            + [pltpu.VMEM((WIN, DIM), jnp.float32) for _ in range(NBUF)]
            + [pltpu.SemaphoreType.DMA for _ in range(2 * NBUF)]
        ),
    )
    def emb_kernel(tok_hbm, i_hbm, pos_hbm, o_hbm, pos_v, idx_v, g0, g1, g2, g3,
                   gs0, gs1, gs2, gs3, ss0, ss1, ss2, ss3):
        bufs = (g0, g1, g2, g3)
        gsems = (gs0, gs1, gs2, gs3)
        ssems = (ss0, ss1, ss2, ss3)

        wid = lax.axis_index("subcore") * NC + lax.axis_index("core")
        base = wid * per_w

        pltpu.sync_copy(i_hbm.at[pl.ds(base, per_w)], idx_v)
        pltpu.sync_copy(pos_hbm, pos_v)

        def gather(w, b):
            pltpu.async_copy(
                tok_hbm.at[idx_v.at[pl.ds(w * WIN, WIN)]], bufs[b], gsems[b]
            )

        def gather_wait(w, b):
            pltpu.make_async_copy(
                tok_hbm.at[idx_v.at[pl.ds(w * WIN, WIN)]], bufs[b], gsems[b]
            ).wait()

        def store(w, b):
            pltpu.async_copy(
                bufs[b], o_hbm.at[pl.ds(base + w * WIN, WIN), :], ssems[b]
            )

        def store_wait(w, b):
            pltpu.make_async_copy(
                bufs[b], o_hbm.at[pl.ds(base + w * WIN, WIN), :], ssems[b]
            ).wait()

        def add_pos(w, b):
            g = bufs[b]
            prow = lax.rem(w, 2) * WIN

            @pl.loop(0, WIN)
            def _(r):
                for j in range(DIM // LANES):
                    slc = pl.ds(j * LANES, LANES)
                    g.at[r, slc][...] = (
                        g.at[r, slc][...] + pos_v.at[prow + r, slc][...]
                    )

        for b in range(NBUF):
            gather(b, b)

        @pl.loop(0, nwin - NBUF, step=NBUF)
        def _(w):
            for b in range(NBUF):
                gather_wait(w + b, b)
                add_pos(w + b, b)
                store(w + b, b)
            for b in range(NBUF):
                store_wait(w + b, b)
                gather(w + NBUF + b, b)

        last = nwin - NBUF
        for b in range(NBUF):
            gather_wait(last + b, b)
            add_pos(last + b, b)
            store(last + b, b)
        for b in range(NBUF):
            store_wait(last + b, b)

    out = emb_kernel(tok_table, idx_flat, pos_table)
    return out.reshape(batch, seq, DIM)


# speedup vs baseline: 2.9548x; 1.0520x over previous
import jax
import jax.numpy as jnp
from jax import lax
from jax.experimental import pallas as pl
from jax.experimental.pallas import tpu as pltpu
from jax.experimental.pallas import tpu_sc as plsc

VOCAB = 30522
SEQ = 128
DIM = 256
LANES = 16
NC = 2
NS = 16
NW = NC * NS
SB = 8
PB = 8
WIN = SB * PB
NBUF = 4


def kernel(x, tok_table, pos_table):
    batch, seq = x.shape
    n = batch * seq
    seq_per_w = batch // NW
    nwin = (seq_per_w // SB) * (seq // PB)
    npb = seq // PB

    x4 = (
        x.astype(jnp.int32)
        .reshape(NW, seq_per_w // SB, SB, npb, PB)
        .transpose(0, 1, 3, 2, 4)
        .reshape(NW, nwin, WIN)
    )

    mesh = plsc.VectorSubcoreMesh(core_axis_name="core", subcore_axis_name="subcore")

    @pl.kernel(
        out_type=jax.ShapeDtypeStruct((n, DIM), jnp.float32),
        mesh=mesh,
        scratch_types=(
            [
                pltpu.VMEM((SEQ, DIM), jnp.float32),
                pltpu.VMEM((nwin, WIN), jnp.int32),
            ]
            + [pltpu.VMEM((WIN, DIM), jnp.float32) for _ in range(NBUF)]
            + [pltpu.SemaphoreType.DMA for _ in range(2 * NBUF)]
        ),
    )
    def emb_kernel(tok_hbm, i_hbm, pos_hbm, o_hbm, pos_v, idx_v, g0, g1, g2, g3,
                   gs0, gs1, gs2, gs3, ss0, ss1, ss2, ss3):
        bufs = (g0, g1, g2, g3)
        gsems = (gs0, gs1, gs2, gs3)
        ssems = (ss0, ss1, ss2, ss3)

        wid = lax.axis_index("subcore") * NC + lax.axis_index("core")

        pltpu.sync_copy(i_hbm.at[wid], idx_v)
        pltpu.sync_copy(pos_hbm, pos_v)

        def gather(w, b):
            pltpu.async_copy(tok_hbm.at[idx_v.at[w]], bufs[b], gsems[b])

        def gather_wait(w, b):
            pltpu.make_async_copy(
                tok_hbm.at[idx_v.at[w]], bufs[b], gsems[b]
            ).wait()

        def _store_slices(w, b, i):
            si = w // npb
            p0 = (w % npb) * PB
            row0 = (wid * seq_per_w + si * SB + i) * seq + p0
            return bufs[b].at[pl.ds(i * PB, PB), :], o_hbm.at[pl.ds(row0, PB), :]

        def store(w, b):
            for i in range(SB):
                src, dst = _store_slices(w, b, i)
                pltpu.async_copy(src, dst, ssems[b])

        def store_wait(w, b):
            for i in range(SB):
                src, dst = _store_slices(w, b, i)
                pltpu.make_async_copy(src, dst, ssems[b]).wait()

        def add_pos(w, b):
            g = bufs[b]
            p0 = (w % npb) * PB

            @pl.loop(0, PB)
            def _(q):
                pv = [
                    pos_v.at[p0 + q, pl.ds(j * LANES, LANES)][...]
                    for j in range(DIM // LANES)
                ]

                @pl.loop(0, SB)
                def _(i):
                    r = i * PB + q
                    for j in range(DIM // LANES):
                        slc = pl.ds(j * LANES, LANES)
                        g.at[r, slc][...] = g.at[r, slc][...] + pv[j]

        for b in range(NBUF):
            gather(b, b)

        @pl.loop(0, nwin - NBUF, step=NBUF)
        def _(w):
            for b in range(NBUF):
                gather_wait(w + b, b)
                add_pos(w + b, b)
                store(w + b, b)
            for b in range(NBUF):
                store_wait(w + b, b)
                gather(w + NBUF + b, b)

        last = nwin - NBUF
        for b in range(NBUF):
            gather_wait(last + b, b)
            add_pos(last + b, b)
            store(last + b, b)
        for b in range(NBUF):
            store_wait(last + b, b)

    out = emb_kernel(tok_table, x4, pos_table)
    return out.reshape(batch, seq, DIM)


# speedup vs baseline: 3.5307x; 1.1949x over previous
import jax
import jax.numpy as jnp
from jax import lax
from jax.experimental import pallas as pl
from jax.experimental.pallas import tpu as pltpu
from jax.experimental.pallas import tpu_sc as plsc

VOCAB = 30522
SEQ = 128
DIM = 256
LANES = 16
NC = 2
NS = 16
NW = NC * NS
SB = 4
PB = 8
WIN = SB * PB
NBUF = 8
LAG = 2


def kernel(x, tok_table, pos_table):
    batch, seq = x.shape
    n = batch * seq
    seq_per_w = batch // NW
    npb = seq // PB
    nwin = (seq_per_w // SB) * npb

    x4 = (
        x.astype(jnp.int32)
        .reshape(NW, seq_per_w // SB, SB, npb, PB)
        .transpose(0, 1, 3, 2, 4)
        .reshape(NW, nwin, WIN)
    )

    mesh = plsc.VectorSubcoreMesh(core_axis_name="core", subcore_axis_name="subcore")

    @pl.kernel(
        out_type=jax.ShapeDtypeStruct((n, DIM), jnp.float32),
        mesh=mesh,
        scratch_types=(
            [
                pltpu.VMEM((SEQ, DIM), jnp.float32),
                pltpu.VMEM((nwin, WIN), jnp.int32),
            ]
            + [pltpu.VMEM((WIN, DIM), jnp.float32) for _ in range(NBUF)]
            + [pltpu.SemaphoreType.DMA for _ in range(2 * NBUF)]
        ),
    )
    def emb_kernel(tok_hbm, i_hbm, pos_hbm, o_hbm, pos_v, idx_v, *rest):
        bufs = rest[:NBUF]
        gsems = rest[NBUF:2 * NBUF]
        ssems = rest[2 * NBUF:]

        wid = lax.axis_index("subcore") * NC + lax.axis_index("core")

        pltpu.sync_copy(i_hbm.at[wid], idx_v)
        pltpu.sync_copy(pos_hbm, pos_v)

        def gather(w, b):
            pltpu.async_copy(tok_hbm.at[idx_v.at[w]], bufs[b], gsems[b])

        def gather_wait(w, b):
            pltpu.make_async_copy(
                tok_hbm.at[idx_v.at[w]], bufs[b], gsems[b]
            ).wait()

        def _store_slices(w, b, i):
            si = w // npb
            p0 = (w % npb) * PB
            row0 = (wid * seq_per_w + si * SB + i) * seq + p0
            return bufs[b].at[pl.ds(i * PB, PB), :], o_hbm.at[pl.ds(row0, PB), :]

        def store(w, b):
            for i in range(SB):
                src, dst = _store_slices(w, b, i)
                pltpu.async_copy(src, dst, ssems[b])

        def store_wait(w, b):
            for i in range(SB):
                src, dst = _store_slices(w, b, i)
                pltpu.make_async_copy(src, dst, ssems[b]).wait()

        def add_pos(w, b):
            g = bufs[b]
            p0 = (w % npb) * PB

            @pl.loop(0, PB)
            def _(q):
                pv = [
                    pos_v.at[p0 + q, pl.ds(j * LANES, LANES)][...]
                    for j in range(DIM // LANES)
                ]

                @pl.loop(0, SB)
                def _(i):
                    r = i * PB + q
                    for j in range(DIM // LANES):
                        slc = pl.ds(j * LANES, LANES)
                        g.at[r, slc][...] = g.at[r, slc][...] + pv[j]

        def process(w, b):
            gather_wait(w, b)
            add_pos(w, b)
            store(w, b)

        for b in range(NBUF):
            gather(b, b)
        for w in range(LAG):
            process(w, w % NBUF)

        @pl.loop(0, nwin - NBUF, step=NBUF)
        def _(w0):
            for b in range(NBUF):
                process(w0 + b + LAG, (b + LAG) % NBUF)
                store_wait(w0 + b, b)
                gather(w0 + b + NBUF, b)

        for k in range(NBUF - LAG):
            w = nwin - NBUF + LAG + k
            process(w, w % NBUF)
        for k in range(NBUF):
            w = nwin - NBUF + k
            store_wait(w, w % NBUF)

    out = emb_kernel(tok_table, x4, pos_table)
    return out.reshape(batch, seq, DIM)


# speedup vs baseline: 4.2957x; 1.2167x over previous
import jax
import jax.numpy as jnp
from jax import lax
from jax.experimental import pallas as pl
from jax.experimental.pallas import tpu as pltpu
from jax.experimental.pallas import tpu_sc as plsc

VOCAB = 30522
SEQ = 128
DIM = 256
LANES = 16
NC = 2
NS = 16
NW = NC * NS
SB = 4
PB = 8
WIN = SB * PB
NBUF = 8
LAG = 2


def kernel(x, tok_table, pos_table):
    batch, seq = x.shape
    n = batch * seq
    seq_per_w = batch // NW
    npb = seq // PB
    nwin = (seq_per_w // SB) * npb

    x4 = (
        x.astype(jnp.int32)
        .reshape(NW, seq_per_w // SB, SB, npb, PB)
        .transpose(0, 1, 3, 2, 4)
        .reshape(NW, nwin, WIN)
    )

    mesh = plsc.VectorSubcoreMesh(core_axis_name="core", subcore_axis_name="subcore")

    @pl.kernel(
        out_type=jax.ShapeDtypeStruct((n, DIM), jnp.float32),
        mesh=mesh,
        scratch_types=(
            [
                pltpu.VMEM((SEQ, DIM), jnp.float32),
                pltpu.VMEM((nwin, WIN), jnp.int32),
            ]
            + [pltpu.VMEM((WIN, DIM), jnp.float32) for _ in range(NBUF)]
            + [pltpu.SemaphoreType.DMA for _ in range(2 * NBUF)]
        ),
    )
    def emb_kernel(tok_hbm, i_hbm, pos_hbm, o_hbm, pos_v, idx_v, *rest):
        bufs = rest[:NBUF]
        gsems = rest[NBUF:2 * NBUF]
        ssems = rest[2 * NBUF:]

        wid = lax.axis_index("subcore") * NC + lax.axis_index("core")

        pltpu.sync_copy(i_hbm.at[wid], idx_v)
        pltpu.sync_copy(pos_hbm, pos_v)

        def gather(w, b):
            pltpu.async_copy(tok_hbm.at[idx_v.at[w]], bufs[b], gsems[b])

        def gather_wait(w, b):
            pltpu.make_async_copy(
                tok_hbm.at[idx_v.at[w]], bufs[b], gsems[b]
            ).wait()

        def _store_slices(w, b, i):
            si = w // npb
            p0 = (w % npb) * PB
            row0 = (wid * seq_per_w + si * SB + i) * seq + p0
            return bufs[b].at[pl.ds(i * PB, PB), :], o_hbm.at[pl.ds(row0, PB), :]

        def store(w, b):
            for i in range(SB):
                src, dst = _store_slices(w, b, i)
                pltpu.async_copy(src, dst, ssems[b])

        def store_wait(w, b):
            pltpu.make_async_copy(
                tok_hbm.at[pl.ds(0, WIN), :], bufs[b], ssems[b]
            ).wait()

        def add_pos(w, b):
            g = bufs[b]
            p0 = (w % npb) * PB

            @pl.loop(0, PB)
            def _(q):
                pv = [
                    pos_v.at[p0 + q, pl.ds(j * LANES, LANES)][...]
                    for j in range(DIM // LANES)
                ]

                @plsc.parallel_loop(0, SB, unroll=SB)
                def _(i):
                    r = i * PB + q
                    for j in range(DIM // LANES):
                        slc = pl.ds(j * LANES, LANES)
                        g.at[r, slc][...] = g.at[r, slc][...] + pv[j]

        def process(w, b):
            gather_wait(w, b)
            add_pos(w, b)
            store(w, b)

        for b in range(NBUF):
            gather(b, b)
        for w in range(LAG):
            process(w, w % NBUF)

        @pl.loop(0, nwin - NBUF, step=NBUF)
        def _(w0):
            for b in range(NBUF):
                process(w0 + b + LAG, (b + LAG) % NBUF)
                store_wait(w0 + b, b)
                gather(w0 + b + NBUF, b)

        for k in range(NBUF - LAG):
            w = nwin - NBUF + LAG + k
            process(w, w % NBUF)
        for k in range(NBUF):
            w = nwin - NBUF + k
            store_wait(w, w % NBUF)

    out = emb_kernel(tok_table, x4, pos_table)
    return out.reshape(batch, seq, DIM)


# speedup vs baseline: 4.3034x; 1.0018x over previous
import jax
import jax.numpy as jnp
from jax import lax
from jax.experimental import pallas as pl
from jax.experimental.pallas import tpu as pltpu
from jax.experimental.pallas import tpu_sc as plsc

VOCAB = 30522
SEQ = 128
DIM = 256
LANES = 16
NC = 2
NS = 16
NW = NC * NS
SB = 4
PB = 8
WIN = SB * PB
NBUF = 8
LAG = 3


def kernel(x, tok_table, pos_table):
    batch, seq = x.shape
    n = batch * seq
    seq_per_w = batch // NW
    npb = seq // PB
    nwin = (seq_per_w // SB) * npb

    x4 = (
        x.astype(jnp.int32)
        .reshape(NW, seq_per_w // SB, SB, npb, PB)
        .transpose(0, 1, 3, 2, 4)
        .reshape(NW, nwin, WIN)
    )

    mesh = plsc.VectorSubcoreMesh(core_axis_name="core", subcore_axis_name="subcore")

    @pl.kernel(
        out_type=jax.ShapeDtypeStruct((n, DIM), jnp.float32),
        mesh=mesh,
        scratch_types=(
            [
                pltpu.VMEM((SEQ, DIM), jnp.float32),
                pltpu.VMEM((nwin, WIN), jnp.int32),
            ]
            + [pltpu.VMEM((WIN, DIM), jnp.float32) for _ in range(NBUF)]
            + [pltpu.SemaphoreType.DMA for _ in range(2 * NBUF)]
        ),
    )
    def emb_kernel(tok_hbm, i_hbm, pos_hbm, o_hbm, pos_v, idx_v, *rest):
        bufs = rest[:NBUF]
        gsems = rest[NBUF:2 * NBUF]
        ssems = rest[2 * NBUF:]

        wid = lax.axis_index("subcore") * NC + lax.axis_index("core")

        pltpu.sync_copy(i_hbm.at[wid], idx_v)
        pltpu.sync_copy(pos_hbm, pos_v)

        def gather(w, b):
            pltpu.async_copy(tok_hbm.at[idx_v.at[w]], bufs[b], gsems[b])

        def gather_wait(w, b):
            pltpu.make_async_copy(
                tok_hbm.at[idx_v.at[w]], bufs[b], gsems[b]
            ).wait()

        def _store_slices(w, b, i):
            si = w // npb
            p0 = (w % npb) * PB
            row0 = (wid * seq_per_w + si * SB + i) * seq + p0
            return bufs[b].at[pl.ds(i * PB, PB), :], o_hbm.at[pl.ds(row0, PB), :]

        def store(w, b):
            for i in range(SB):
                src, dst = _store_slices(w, b, i)
                pltpu.async_copy(src, dst, ssems[b])

        def store_wait(w, b):
            pltpu.make_async_copy(
                tok_hbm.at[pl.ds(0, WIN), :], bufs[b], ssems[b]
            ).wait()

        def add_pos(w, b):
            g = bufs[b]
            p0 = (w % npb) * PB

            @pl.loop(0, PB)
            def _(q):
                pv = [
                    pos_v.at[p0 + q, pl.ds(j * LANES, LANES)][...]
                    for j in range(DIM // LANES)
                ]

                @plsc.parallel_loop(0, SB, unroll=SB)
                def _(i):
                    r = i * PB + q
                    for j in range(DIM // LANES):
                        slc = pl.ds(j * LANES, LANES)
                        g.at[r, slc][...] = g.at[r, slc][...] + pv[j]

        def process(w, b):
            gather_wait(w, b)
            add_pos(w, b)
            store(w, b)

        for b in range(NBUF):
            gather(b, b)
        for w in range(LAG):
            process(w, w % NBUF)

        @pl.loop(0, nwin - NBUF, step=NBUF)
        def _(w0):
            for b in range(NBUF):
                process(w0 + b + LAG, (b + LAG) % NBUF)
                store_wait(w0 + b, b)
                gather(w0 + b + NBUF, b)

        for k in range(NBUF - LAG):
            w = nwin - NBUF + LAG + k
            process(w, w % NBUF)
        for k in range(NBUF):
            w = nwin - NBUF + k
            store_wait(w, w % NBUF)

    out = emb_kernel(tok_table, x4, pos_table)
    return out.reshape(batch, seq, DIM)
